# Initial kernel scaffold; baseline (speedup 1.0000x reference)
import jax
import jax.numpy as jnp
from jax.experimental import pallas as pl


def kernel(x, table, W, b):
    raise NotImplementedError("write your pallas kernel here")


# speedup vs baseline: 7.6496x; 7.6496x over previous
import functools

import jax
import jax.numpy as jnp
from jax import lax
from jax.experimental import pallas as pl
from jax.experimental.pallas import tpu as pltpu
from jax.experimental.pallas import tpu_sc as plsc

B = 16384
N_CONT = 13
N_CAT = 26
CARD = 100000
EMB = 16
XW = 40

L = 16
NC = 2
NS = 16
NW = NC * NS
RPW = B // NW
CH = 128
NCHUNK = RPW // CH

_mesh = plsc.VectorSubcoreMesh(core_axis_name="c", subcore_axis_name="s")


@functools.partial(
    pl.kernel,
    mesh=_mesh,
    compiler_params=pltpu.CompilerParams(
        needs_layout_passes=False, use_tc_tiling_on_sc=False
    ),
    out_type=jax.ShapeDtypeStruct((B,), jnp.float32),
    scratch_types=[
        pltpu.VMEM((CH * XW,), jnp.float32),
        pltpu.VMEM((N_CAT, CH), jnp.int32),
        pltpu.VMEM((N_CAT, CH, EMB), jnp.float32),
        pltpu.VMEM((L * L,), jnp.float32),
        pltpu.VMEM((N_CAT + 2, L), jnp.float32),
        pltpu.VMEM((RPW,), jnp.float32),
        pltpu.SemaphoreType.DMA,
    ],
)
def _sc_logits(x_hbm, table_hbm, w_hbm, out_hbm, xc, idxv, rows, accb, wv, outv, sem):
    wid = lax.axis_index("s") * NC + lax.axis_index("c")
    wbase = wid * RPW

    pltpu.sync_copy(w_hbm, wv)
    iota = lax.iota(jnp.int32, L)
    iota_row = iota * XW
    wc = wv[0, :]
    wj = [wv[1 + j, :] for j in range(N_CAT)]
    bvec = wv[N_CAT + 1, :]

    def chunk_body(c, _):
        row0 = wbase + c * CH
        pltpu.sync_copy(x_hbm.at[pl.ds(row0 * XW, CH * XW)], xc)

        def field_body(j, _):
            base = j * CARD
            col = N_CONT + j
            for s in range(CH // L):
                vals = plsc.load_gather(xc, [iota_row + (s * L * XW + col)])
                idxv[j, pl.ds(s * L, L)] = vals.astype(jnp.int32) + base
            pltpu.async_copy(table_hbm.at[idxv.at[j]], rows.at[j], sem)
            return _

        lax.fori_loop(0, N_CAT, field_body, None)

        def drain_body(j, _):
            pltpu.make_async_copy(table_hbm.at[idxv.at[j]], rows.at[j], sem).wait()
            return _

        lax.fori_loop(0, N_CAT, drain_body, None)

        def sub_body(s, _):
            for di in range(L):
                i = s * L + di
                xrow = plsc.load_gather(xc, [iota + i * XW])
                acc = xrow * wc
                for j in range(N_CAT):
                    acc = acc + rows[j, i, :] * wj[j]
                accb[pl.ds(di * L, L)] = acc
            res = bvec
            for d in range(L):
                res = res + plsc.load_gather(accb, [iota * L + d])
            outv[pl.ds(c * CH + s * L, L)] = res
            return _

        lax.fori_loop(0, CH // L, sub_body, None)
        return _

    lax.fori_loop(0, NCHUNK, chunk_body, None)
    pltpu.sync_copy(outv, out_hbm.at[pl.ds(wbase, RPW)])


def kernel(x, table, W, b):
    xpad = jnp.pad(x, ((0, 0), (0, XW - x.shape[1]))).reshape(-1)
    wcont = jnp.pad(W[:N_CONT, 0], (0, L - N_CONT))
    wemb = W[N_CONT:, 0].reshape(N_CAT, EMB)
    brow = jnp.broadcast_to(b.reshape(1, 1), (1, L))
    wall = jnp.concatenate([wcont[None, :], wemb, brow], axis=0)
    out = _sc_logits(xpad, table, wall)
    return out.reshape(B, 1)


# speedup vs baseline: 63.0209x; 8.2385x over previous
import functools

import jax
import jax.numpy as jnp
from jax import lax
from jax.experimental import pallas as pl
from jax.experimental.pallas import tpu as pltpu
from jax.experimental.pallas import tpu_sc as plsc

B = 16384
N_CONT = 13
N_CAT = 26
CARD = 100000
EMB = 16
XW = 40
NTAB = N_CAT * CARD

L = 16
NC = 2
NS = 16
NW = NC * NS
RPW = B // NW
NBLK = RPW // L
NH = RPW // 128

CBLK = 65536
TC_GRID = (NTAB + CBLK - 1) // CBLK


def _tc_scale_body(tt_ref, wt_ref, s_ref):
    col0 = pl.program_id(0) * CBLK
    f0 = col0 // CARD
    fi = lax.broadcasted_iota(jnp.int32, (N_CAT, 2), 0)
    tgt = f0 + lax.broadcasted_iota(jnp.int32, (N_CAT, 2), 1)
    oh = (fi == tgt).astype(jnp.float32)
    w01 = jnp.dot(wt_ref[:], oh, preferred_element_type=jnp.float32)
    colid = col0 + lax.broadcasted_iota(jnp.int32, (EMB, CBLK), 1)
    w = jnp.where(colid >= (f0 + 1) * CARD, w01[:, 1:2], w01[:, 0:1])
    s_ref[:] = jnp.sum(tt_ref[:] * w, axis=0)


_tc_scale = pl.pallas_call(
    _tc_scale_body,
    grid=(TC_GRID,),
    in_specs=[
        pl.BlockSpec((EMB, CBLK), lambda i: (0, i)),
        pl.BlockSpec((EMB, N_CAT), lambda i: (0, 0)),
    ],
    out_specs=pl.BlockSpec((CBLK,), lambda i: (i,)),
    out_shape=jax.ShapeDtypeStruct((NTAB,), jnp.float32),
    compiler_params=pltpu.CompilerParams(
        dimension_semantics=("arbitrary",),
    ),
)

_mesh = plsc.VectorSubcoreMesh(core_axis_name="c", subcore_axis_name="s")


@functools.partial(
    pl.kernel,
    mesh=_mesh,
    compiler_params=pltpu.CompilerParams(
        needs_layout_passes=False, use_tc_tiling_on_sc=False
    ),
    out_type=jax.ShapeDtypeStruct((B,), jnp.float32),
    scratch_types=[
        pltpu.VMEM((RPW * XW,), jnp.float32),
        pltpu.VMEM((N_CAT, NH, 128), jnp.int32),
        pltpu.VMEM((N_CAT, NH, 128), jnp.float32),
        pltpu.VMEM((2, L), jnp.float32),
        pltpu.VMEM((RPW,), jnp.float32),
        pltpu.SemaphoreType.DMA,
    ],
)
def _sc_logits(x_hbm, s_hbm, w_hbm, out_hbm, xc, idxv, sv, wv, outv, sem):
    wid = lax.axis_index("s") * NC + lax.axis_index("c")
    wbase = wid * RPW

    pltpu.sync_copy(w_hbm, wv)
    pltpu.sync_copy(x_hbm.at[pl.ds(wbase * XW, RPW * XW)], xc)
    iota = lax.iota(jnp.int32, L)
    iota_row = iota * XW

    def field_body(j, _):
        base = j * CARD
        col = N_CONT + j
        for t in range(NBLK):
            vals = plsc.load_gather(xc, [iota_row + (t * L * XW + col)])
            fi = vals.astype(jnp.int32) + base
            idxv[j, t // 8, pl.ds((t % 8) * L, L)] = fi
        for h in range(NH):
            pltpu.async_copy(s_hbm.at[idxv.at[j, h]], sv.at[j, h], sem)
        return _

    lax.fori_loop(0, N_CAT, field_body, None)

    def drain_body(j, _):
        for h in range(NH):
            pltpu.make_async_copy(s_hbm.at[idxv.at[j, h]], sv.at[j, h], sem).wait()
        return _

    lax.fori_loop(0, N_CAT, drain_body, None)

    wrow = wv[0, :]
    wcont = [wrow[c] for c in range(N_CONT)]
    bvec = wv[1, :]

    def block_body(t, _):
        acc = bvec
        xoff = t * L * XW
        for c in range(N_CONT):
            acc = acc + plsc.load_gather(xc, [iota_row + (xoff + c)]) * wcont[c]
        h = t // 8
        off = (t % 8) * L
        for j in range(N_CAT):
            acc = acc + sv[j, h, pl.ds(off, L)]
        outv[pl.ds(t * L, L)] = acc
        return _

    lax.fori_loop(0, NBLK, block_body, None)
    pltpu.sync_copy(outv, out_hbm.at[pl.ds(wbase, RPW)])


def kernel(x, table, W, b):
    xflat = jnp.pad(x, ((0, 0), (0, XW - x.shape[1]))).reshape(-1)
    wt = W[N_CONT:, 0].reshape(N_CAT, EMB).T
    s = _tc_scale(table.T, wt)
    wcont = jnp.pad(W[:N_CONT, 0], (0, L - N_CONT))
    brow = jnp.broadcast_to(b.reshape(1, 1), (1, L))
    wv = jnp.concatenate([wcont[None, :], brow], axis=0)
    out = _sc_logits(xflat, s, wv)
    return out.reshape(B, 1)


# speedup vs baseline: 71.8561x; 1.1402x over previous
import functools

import jax
import jax.numpy as jnp
from jax import lax
from jax.experimental import pallas as pl
from jax.experimental.pallas import tpu as pltpu
from jax.experimental.pallas import tpu_sc as plsc

B = 16384
N_CONT = 13
N_CAT = 26
CARD = 100000
EMB = 16
XW = 40
NTAB = N_CAT * CARD

L = 16
NC = 2
NS = 16
NW = NC * NS
RPW = B // NW
NBLK = RPW // L
NH = RPW // 128

CBLK = 65536
TC_GRID = (NTAB + CBLK - 1) // CBLK


def _tc_scale_body(tt_ref, wt_ref, s_ref):
    col0 = pl.program_id(0) * CBLK
    f0 = col0 // CARD
    fi = lax.broadcasted_iota(jnp.int32, (2, N_CAT), 1)
    tgt = f0 + lax.broadcasted_iota(jnp.int32, (2, N_CAT), 0)
    oh = (fi == tgt).astype(jnp.float32)
    w01 = jax.lax.dot_general(
        oh, wt_ref[:], (((1,), (1,)), ((), ())),
        preferred_element_type=jnp.float32,
    )
    r = jax.lax.dot_general(
        w01, tt_ref[:], (((1,), (0,)), ((), ())),
        preferred_element_type=jnp.float32,
    )
    colrel = lax.broadcasted_iota(jnp.int32, (1, CBLK), 1)
    use1 = colrel >= (f0 + 1) * CARD - col0
    s_ref[:] = jnp.where(use1, r[1:2, :], r[0:1, :])[0]


_tc_scale = pl.pallas_call(
    _tc_scale_body,
    grid=(TC_GRID,),
    in_specs=[
        pl.BlockSpec((EMB, CBLK), lambda i: (0, i)),
        pl.BlockSpec((EMB, N_CAT), lambda i: (0, 0)),
    ],
    out_specs=pl.BlockSpec((CBLK,), lambda i: (i,)),
    out_shape=jax.ShapeDtypeStruct((NTAB,), jnp.float32),
    compiler_params=pltpu.CompilerParams(
        dimension_semantics=("arbitrary",),
    ),
)

_mesh = plsc.VectorSubcoreMesh(core_axis_name="c", subcore_axis_name="s")


@functools.partial(
    pl.kernel,
    mesh=_mesh,
    compiler_params=pltpu.CompilerParams(
        needs_layout_passes=False, use_tc_tiling_on_sc=False
    ),
    out_type=jax.ShapeDtypeStruct((B,), jnp.float32),
    scratch_types=[
        pltpu.VMEM((RPW * XW,), jnp.float32),
        pltpu.VMEM((N_CAT, NH, 128), jnp.int32),
        pltpu.VMEM((N_CAT, NH, 128), jnp.float32),
        pltpu.VMEM((2, L), jnp.float32),
        pltpu.VMEM((RPW,), jnp.float32),
        pltpu.SemaphoreType.DMA,
    ],
)
def _sc_logits(x_hbm, s_hbm, w_hbm, out_hbm, xc, idxv, sv, wv, outv, sem):
    wid = lax.axis_index("s") * NC + lax.axis_index("c")
    wbase = wid * RPW

    pltpu.sync_copy(w_hbm, wv)
    pltpu.sync_copy(x_hbm.at[pl.ds(wbase * XW, RPW * XW)], xc)
    iota = lax.iota(jnp.int32, L)
    iota_row = iota * XW

    def field_body(j, _):
        base = j * CARD
        col = N_CONT + j
        for t in range(NBLK):
            vals = plsc.load_gather(xc, [iota_row + (t * L * XW + col)])
            fi = vals.astype(jnp.int32) + base
            idxv[j, t // 8, pl.ds((t % 8) * L, L)] = fi
        for h in range(NH):
            pltpu.async_copy(s_hbm.at[idxv.at[j, h]], sv.at[j, h], sem)
        return _

    lax.fori_loop(0, N_CAT, field_body, None)

    def drain_body(j, _):
        for h in range(NH):
            pltpu.make_async_copy(s_hbm.at[idxv.at[j, h]], sv.at[j, h], sem).wait()
        return _

    lax.fori_loop(0, N_CAT, drain_body, None)

    wrow = wv[0, :]
    wcont = [wrow[c] for c in range(N_CONT)]
    bvec = wv[1, :]

    def block_body(t, _):
        acc = bvec
        xoff = t * L * XW
        for c in range(N_CONT):
            acc = acc + plsc.load_gather(xc, [iota_row + (xoff + c)]) * wcont[c]
        h = t // 8
        off = (t % 8) * L
        for j in range(N_CAT):
            acc = acc + sv[j, h, pl.ds(off, L)]
        outv[pl.ds(t * L, L)] = acc
        return _

    lax.fori_loop(0, NBLK, block_body, None)
    pltpu.sync_copy(outv, out_hbm.at[pl.ds(wbase, RPW)])


def kernel(x, table, W, b):
    xflat = jnp.pad(x, ((0, 0), (0, XW - x.shape[1]))).reshape(-1)
    wt = W[N_CONT:, 0].reshape(N_CAT, EMB).T
    s = _tc_scale(table.T, wt)
    wcont = jnp.pad(W[:N_CONT, 0], (0, L - N_CONT))
    brow = jnp.broadcast_to(b.reshape(1, 1), (1, L))
    wv = jnp.concatenate([wcont[None, :], brow], axis=0)
    out = _sc_logits(xflat, s, wv)
    return out.reshape(B, 1)


# speedup vs baseline: 76.9679x; 1.0711x over previous
import functools

import jax
import jax.numpy as jnp
from jax import lax
from jax.experimental import pallas as pl
from jax.experimental.pallas import tpu as pltpu
from jax.experimental.pallas import tpu_sc as plsc

B = 16384
N_CONT = 13
N_CAT = 26
CARD = 100000
EMB = 16
XW = 40
NTAB = N_CAT * CARD

L = 16
NC = 2
NS = 16
NW = NC * NS
RPW = B // NW
NBLK = RPW // L
NH = RPW // 128

CBLK = 98304
TC_GRID = (NTAB + CBLK - 1) // CBLK


def _tc_scale_body(tt_ref, wt_ref, s_ref):
    col0 = pl.program_id(0) * CBLK
    f0 = col0 // CARD
    fi = lax.broadcasted_iota(jnp.int32, (2, N_CAT), 1)
    tgt = f0 + lax.broadcasted_iota(jnp.int32, (2, N_CAT), 0)
    oh = (fi == tgt).astype(jnp.float32)
    w01 = jax.lax.dot_general(
        oh, wt_ref[:], (((1,), (1,)), ((), ())),
        preferred_element_type=jnp.float32,
    )
    r = jax.lax.dot_general(
        w01, tt_ref[:], (((1,), (0,)), ((), ())),
        preferred_element_type=jnp.float32,
    )
    colrel = lax.broadcasted_iota(jnp.int32, (1, CBLK), 1)
    use1 = colrel >= (f0 + 1) * CARD - col0
    s_ref[:] = jnp.where(use1, r[1:2, :], r[0:1, :])[0]


_tc_scale = pl.pallas_call(
    _tc_scale_body,
    grid=(TC_GRID,),
    in_specs=[
        pl.BlockSpec((EMB, CBLK), lambda i: (0, i)),
        pl.BlockSpec((EMB, N_CAT), lambda i: (0, 0)),
    ],
    out_specs=pl.BlockSpec((CBLK,), lambda i: (i,)),
    out_shape=jax.ShapeDtypeStruct((NTAB,), jnp.float32),
    compiler_params=pltpu.CompilerParams(
        dimension_semantics=("arbitrary",),
    ),
)

_mesh = plsc.VectorSubcoreMesh(core_axis_name="c", subcore_axis_name="s")


@functools.partial(
    pl.kernel,
    mesh=_mesh,
    compiler_params=pltpu.CompilerParams(
        needs_layout_passes=False, use_tc_tiling_on_sc=False
    ),
    out_type=jax.ShapeDtypeStruct((B,), jnp.float32),
    scratch_types=[
        pltpu.VMEM((RPW * XW,), jnp.float32),
        pltpu.VMEM((N_CAT, NH, 128), jnp.int32),
        pltpu.VMEM((N_CAT, NH, 128), jnp.float32),
        pltpu.VMEM((2, L), jnp.float32),
        pltpu.VMEM((RPW,), jnp.float32),
        pltpu.SemaphoreType.DMA,
    ],
)
def _sc_logits(x_hbm, s_hbm, w_hbm, out_hbm, xc, idxv, sv, wv, outv, sem):
    wid = lax.axis_index("s") * NC + lax.axis_index("c")
    wbase = wid * RPW

    pltpu.sync_copy(w_hbm, wv)
    pltpu.sync_copy(x_hbm.at[pl.ds(wbase * XW, RPW * XW)], xc)
    iota = lax.iota(jnp.int32, L)
    iota_row = iota * XW

    def field_body(j, _):
        base = j * CARD
        col = N_CONT + j
        for t in range(NBLK):
            vals = plsc.load_gather(xc, [iota_row + (t * L * XW + col)])
            fi = vals.astype(jnp.int32) + base
            idxv[j, t // 8, pl.ds((t % 8) * L, L)] = fi
        for h in range(NH):
            pltpu.async_copy(s_hbm.at[idxv.at[j, h]], sv.at[j, h], sem)
        return _

    lax.fori_loop(0, N_CAT, field_body, None)

    def drain_body(j, _):
        for h in range(NH):
            pltpu.make_async_copy(s_hbm.at[idxv.at[j, h]], sv.at[j, h], sem).wait()
        return _

    lax.fori_loop(0, N_CAT, drain_body, None)

    wrow = wv[0, :]
    wcont = [wrow[c] for c in range(N_CONT)]
    bvec = wv[1, :]

    def block_body(t, _):
        acc = bvec
        xoff = t * L * XW
        for c in range(N_CONT):
            acc = acc + plsc.load_gather(xc, [iota_row + (xoff + c)]) * wcont[c]
        h = t // 8
        off = (t % 8) * L
        for j in range(N_CAT):
            acc = acc + sv[j, h, pl.ds(off, L)]
        outv[pl.ds(t * L, L)] = acc
        return _

    lax.fori_loop(0, NBLK, block_body, None)
    pltpu.sync_copy(outv, out_hbm.at[pl.ds(wbase, RPW)])


def kernel(x, table, W, b):
    xflat = jnp.pad(x, ((0, 0), (0, XW - x.shape[1]))).reshape(-1)
    wt = W[N_CONT:, 0].reshape(N_CAT, EMB).T
    s = _tc_scale(table.T, wt)
    wcont = jnp.pad(W[:N_CONT, 0], (0, L - N_CONT))
    brow = jnp.broadcast_to(b.reshape(1, 1), (1, L))
    wv = jnp.concatenate([wcont[None, :], brow], axis=0)
    out = _sc_logits(xflat, s, wv)
    return out.reshape(B, 1)
